# baseline (device time: 244942 ns/iter reference)
import jax
import jax.numpy as jnp
from jax import lax
from jax.experimental import pallas as pl
from jax.experimental.pallas import tpu as pltpu

N_DEV = 32


def kernel(x, router_W, route_idx, expert_W):
    n, d = x.shape
    n_exp = router_W.shape[1]
    e_local = expert_W.shape[0]
    h = expert_W.shape[2]

    def body(x_ref, rw_ref, idx_ref, ew_ref, out_ref,
             comm_ref, send_sems, recv_sems):
        my = lax.axis_index("i")
        left = lax.rem(my - 1 + N_DEV, N_DEV)
        right = lax.rem(my + 1, N_DEV)

        barrier = pltpu.get_barrier_semaphore()
        pl.semaphore_signal(barrier, inc=1, device_id=(left,),
                            device_id_type=pl.DeviceIdType.MESH)
        pl.semaphore_signal(barrier, inc=1, device_id=(right,),
                            device_id_type=pl.DeviceIdType.MESH)
        pl.semaphore_wait(barrier, 2)

        xf = x_ref[:, :]
        scores = jnp.dot(xf, rw_ref[:, :],
                         preferred_element_type=jnp.float32)
        smax = jnp.max(scores, axis=1, keepdims=True)
        p = jnp.exp(scores - smax)
        probs = p / jnp.sum(p, axis=1, keepdims=True)

        idx0 = idx_ref[:, 0:1]
        idx1 = idx_ref[:, 1:2]
        cols = lax.broadcasted_iota(jnp.int32, (n, n_exp), 1)
        g0 = jnp.sum(jnp.where(cols == idx0, probs, 0.0), axis=1,
                     keepdims=True)
        g1 = jnp.sum(jnp.where(cols == idx1, probs, 0.0), axis=1,
                     keepdims=True)
        gs = g0 + g1

        acc = jnp.zeros((n, h), jnp.float32)
        for j in range(e_local):
            e = my * e_local + j
            p_e = jnp.sum(jnp.where(cols == e, probs, 0.0), axis=1,
                          keepdims=True)
            mask = jnp.logical_or(idx0 == e, idx1 == e)
            g_e = jnp.where(mask, p_e / gs, 0.0)
            xg = (xf * g_e).astype(jnp.bfloat16)
            acc = acc + jnp.dot(xg, ew_ref[j].astype(jnp.bfloat16),
                                preferred_element_type=jnp.float32)

        out_ref[:, :] = acc
        comm_ref[0] = acc.astype(jnp.bfloat16)

        for k in range(N_DEV - 1):
            rdma = pltpu.make_async_remote_copy(
                src_ref=comm_ref.at[k],
                dst_ref=comm_ref.at[k + 1],
                send_sem=send_sems.at[k],
                recv_sem=recv_sems.at[k],
                device_id=(right,),
                device_id_type=pl.DeviceIdType.MESH,
            )
            rdma.start()
            rdma.wait()
            out_ref[:, :] = out_ref[:, :] + comm_ref[k + 1].astype(jnp.float32)

    return pl.pallas_call(
        body,
        out_shape=jax.ShapeDtypeStruct((n, h), jnp.float32),
        in_specs=[pl.BlockSpec(memory_space=pltpu.VMEM)] * 4,
        out_specs=pl.BlockSpec(memory_space=pltpu.VMEM),
        scratch_shapes=[
            pltpu.VMEM((N_DEV, n, h), jnp.bfloat16),
            pltpu.SemaphoreType.DMA((N_DEV - 1,)),
            pltpu.SemaphoreType.DMA((N_DEV - 1,)),
        ],
        compiler_params=pltpu.CompilerParams(collective_id=0),
    )(x, router_W, route_idx, expert_W)


# device time: 41186 ns/iter; 5.9472x vs baseline; 5.9472x over previous
import jax
import jax.numpy as jnp
from jax import lax
from jax.experimental import pallas as pl
from jax.experimental.pallas import tpu as pltpu

N_DEV = 32
LOG_N = 5

RS_LEN = [512 >> (k + 1) for k in range(LOG_N)]
RS_OFF = [0, 256, 384, 448, 480]
AG_OFF = {4: 512, 3: 528, 2: 560, 1: 624, 0: 752}


def kernel(x, router_W, route_idx, expert_W):
    n, d = x.shape
    n_exp = router_W.shape[1]
    e_local = expert_W.shape[0]
    h = expert_W.shape[2]

    def body(x_ref, rw_ref, idx_ref, ew_ref, out_ref,
             stage_ref, recv_ref, send_sems, recv_sems):
        my = lax.axis_index("i")

        barrier = pltpu.get_barrier_semaphore()
        for k in range(LOG_N):
            pl.semaphore_signal(barrier, inc=1, device_id=(my ^ (1 << k),),
                                device_id_type=pl.DeviceIdType.MESH)
        pl.semaphore_wait(barrier, LOG_N)

        xf = x_ref[:, :]
        scores = jnp.dot(xf, rw_ref[:, :],
                         preferred_element_type=jnp.float32)
        smax = jnp.max(scores, axis=1, keepdims=True)
        p = jnp.exp(scores - smax)
        probs = p / jnp.sum(p, axis=1, keepdims=True)

        idx0 = idx_ref[:, 0:1]
        idx1 = idx_ref[:, 1:2]
        cols = lax.broadcasted_iota(jnp.int32, (n, n_exp), 1)
        g0 = jnp.sum(jnp.where(cols == idx0, probs, 0.0), axis=1,
                     keepdims=True)
        g1 = jnp.sum(jnp.where(cols == idx1, probs, 0.0), axis=1,
                     keepdims=True)
        gs = g0 + g1

        acc = jnp.zeros((n, h), jnp.float32)
        for j in range(e_local):
            e = my * e_local + j
            p_e = jnp.sum(jnp.where(cols == e, probs, 0.0), axis=1,
                          keepdims=True)
            mask = jnp.logical_or(idx0 == e, idx1 == e)
            g_e = jnp.where(mask, p_e / gs, 0.0)
            xg = (xf * g_e).astype(jnp.bfloat16)
            acc = acc + jnp.dot(xg, ew_ref[j].astype(jnp.bfloat16),
                                preferred_element_type=jnp.float32)
        out_ref[:, :] = acc

        off = my * 0
        for k in range(LOG_N):
            half = RS_LEN[k]
            partner = my ^ (1 << k)
            bit = (my >> k) & 1
            send_off = off + (1 - bit) * half
            keep_off = off + bit * half

            stage_ref[pl.ds(RS_OFF[k], half), :] = (
                out_ref[pl.ds(send_off, half), :].astype(jnp.bfloat16))
            rdma = pltpu.make_async_remote_copy(
                src_ref=stage_ref.at[pl.ds(RS_OFF[k], half)],
                dst_ref=recv_ref.at[pl.ds(RS_OFF[k], half)],
                send_sem=send_sems.at[k],
                recv_sem=recv_sems.at[k],
                device_id=(partner,),
                device_id_type=pl.DeviceIdType.MESH,
            )
            rdma.start()
            rdma.wait()
            out_ref[pl.ds(keep_off, half), :] = (
                out_ref[pl.ds(keep_off, half), :]
                + recv_ref[pl.ds(RS_OFF[k], half), :].astype(jnp.float32))
            off = keep_off

        for k in reversed(range(LOG_N)):
            cur_len = 512 >> (k + 1)
            partner = my ^ (1 << k)
            bit = (my >> k) & 1
            partner_off = off + (1 - 2 * bit) * cur_len

            stage_ref[pl.ds(AG_OFF[k], cur_len), :] = (
                out_ref[pl.ds(off, cur_len), :].astype(jnp.bfloat16))
            rdma = pltpu.make_async_remote_copy(
                src_ref=stage_ref.at[pl.ds(AG_OFF[k], cur_len)],
                dst_ref=recv_ref.at[pl.ds(AG_OFF[k], cur_len)],
                send_sem=send_sems.at[LOG_N + 4 - k],
                recv_sem=recv_sems.at[LOG_N + 4 - k],
                device_id=(partner,),
                device_id_type=pl.DeviceIdType.MESH,
            )
            rdma.start()
            rdma.wait()
            out_ref[pl.ds(partner_off, cur_len), :] = (
                recv_ref[pl.ds(AG_OFF[k], cur_len), :].astype(jnp.float32))
            off = off - bit * cur_len

    return pl.pallas_call(
        body,
        out_shape=jax.ShapeDtypeStruct((n, h), jnp.float32),
        in_specs=[pl.BlockSpec(memory_space=pltpu.VMEM)] * 4,
        out_specs=pl.BlockSpec(memory_space=pltpu.VMEM),
        scratch_shapes=[
            pltpu.VMEM((1024, h), jnp.bfloat16),
            pltpu.VMEM((1024, h), jnp.bfloat16),
            pltpu.SemaphoreType.DMA((2 * LOG_N,)),
            pltpu.SemaphoreType.DMA((2 * LOG_N,)),
        ],
        compiler_params=pltpu.CompilerParams(collective_id=0),
    )(x, router_W, route_idx, expert_W)


# device time: 31815 ns/iter; 7.6989x vs baseline; 1.2945x over previous
import jax
import jax.numpy as jnp
from jax import lax
from jax.experimental import pallas as pl
from jax.experimental.pallas import tpu as pltpu

N_DEV = 32
C = 16
AG_STAGE = 512
AG_RECV = 496


def kernel(x, router_W, route_idx, expert_W):
    n, d = x.shape
    n_exp = router_W.shape[1]
    e_local = expert_W.shape[0]
    h = expert_W.shape[2]

    def body(x_ref, rw_ref, idx_ref, ew_ref, out_ref,
             stage_ref, recv_ref, send_sems, recv_sems):
        my = lax.axis_index("i")

        barrier = pltpu.get_barrier_semaphore()
        for r in range(1, N_DEV):
            pl.semaphore_signal(barrier, inc=1,
                                device_id=(lax.rem(my + r, N_DEV),),
                                device_id_type=pl.DeviceIdType.MESH)
        pl.semaphore_wait(barrier, N_DEV - 1)

        xf = x_ref[:, :]
        scores = jnp.dot(xf, rw_ref[:, :],
                         preferred_element_type=jnp.float32)
        smax = jnp.max(scores, axis=1, keepdims=True)
        p = jnp.exp(scores - smax)
        probs = p / jnp.sum(p, axis=1, keepdims=True)

        idx0 = idx_ref[:, 0:1]
        idx1 = idx_ref[:, 1:2]
        cols = lax.broadcasted_iota(jnp.int32, (n, n_exp), 1)
        g0 = jnp.sum(jnp.where(cols == idx0, probs, 0.0), axis=1,
                     keepdims=True)
        g1 = jnp.sum(jnp.where(cols == idx1, probs, 0.0), axis=1,
                     keepdims=True)
        gs = g0 + g1

        acc = jnp.zeros((n, h), jnp.float32)
        for j in range(e_local):
            e = my * e_local + j
            p_e = jnp.sum(jnp.where(cols == e, probs, 0.0), axis=1,
                          keepdims=True)
            mask = jnp.logical_or(idx0 == e, idx1 == e)
            g_e = jnp.where(mask, p_e / gs, 0.0)
            xg = (xf * g_e).astype(jnp.bfloat16)
            acc = acc + jnp.dot(xg, ew_ref[j].astype(jnp.bfloat16),
                                preferred_element_type=jnp.float32)
        out_ref[:, :] = acc
        stage_ref[pl.ds(0, n), :] = acc.astype(jnp.bfloat16)

        rs = []
        for r in range(1, N_DEV):
            t = lax.rem(my + r, N_DEV)
            rdma = pltpu.make_async_remote_copy(
                src_ref=stage_ref.at[pl.ds(t * C, C)],
                dst_ref=recv_ref.at[pl.ds((r - 1) * C, C)],
                send_sem=send_sems.at[r - 1],
                recv_sem=recv_sems.at[r - 1],
                device_id=(t,),
                device_id_type=pl.DeviceIdType.MESH,
            )
            rdma.start()
            rs.append(rdma)

        red = out_ref[pl.ds(my * C, C), :]
        for r in range(1, N_DEV):
            rs[r - 1].wait_recv()
            red = red + recv_ref[pl.ds((r - 1) * C, C), :].astype(jnp.float32)
        out_ref[pl.ds(my * C, C), :] = red
        stage_ref[pl.ds(AG_STAGE, C), :] = red.astype(jnp.bfloat16)

        ag = []
        for r in range(1, N_DEV):
            t = lax.rem(my + r, N_DEV)
            rdma = pltpu.make_async_remote_copy(
                src_ref=stage_ref.at[pl.ds(AG_STAGE, C)],
                dst_ref=recv_ref.at[pl.ds(AG_RECV + (r - 1) * C, C)],
                send_sem=send_sems.at[N_DEV - 1 + r - 1],
                recv_sem=recv_sems.at[N_DEV - 1 + r - 1],
                device_id=(t,),
                device_id_type=pl.DeviceIdType.MESH,
            )
            rdma.start()
            ag.append(rdma)

        for r in range(1, N_DEV):
            ag[r - 1].wait_recv()
            origin = lax.rem(my - r + N_DEV, N_DEV)
            out_ref[pl.ds(origin * C, C), :] = (
                recv_ref[pl.ds(AG_RECV + (r - 1) * C, C), :]
                .astype(jnp.float32))

        for rdma in rs:
            rdma.wait_send()
        for rdma in ag:
            rdma.wait_send()

    return pl.pallas_call(
        body,
        out_shape=jax.ShapeDtypeStruct((n, h), jnp.float32),
        in_specs=[pl.BlockSpec(memory_space=pltpu.VMEM)] * 4,
        out_specs=pl.BlockSpec(memory_space=pltpu.VMEM),
        scratch_shapes=[
            pltpu.VMEM((528, h), jnp.bfloat16),
            pltpu.VMEM((992, h), jnp.bfloat16),
            pltpu.SemaphoreType.DMA((2 * (N_DEV - 1),)),
            pltpu.SemaphoreType.DMA((2 * (N_DEV - 1),)),
        ],
        compiler_params=pltpu.CompilerParams(collective_id=0),
    )(x, router_W, route_idx, expert_W)
